# baseline (speedup 1.0000x reference)
import functools

import jax
import jax.numpy as jnp
from jax import lax
from jax.experimental import pallas as pl
from jax.experimental.pallas import tpu as pltpu
from jax.experimental.pallas import tpu_sc as plsc

G = 128
GGG = G * G * G
NCHAN = 5
CHUNK = 128
NGROUPS = CHUNK // 16
NBUF = 2
CELLS = 2048

_SC_PARAMS = pltpu.CompilerParams(
    needs_layout_passes=False, use_tc_tiling_on_sc=False)


def _build_table(hflat, eflat):
    info = plsc.get_sparse_core_info()
    nw = info.num_cores * info.num_subcores
    per_tile = GGG // nw
    n_chunks = per_tile // CELLS

    mesh = plsc.VectorSubcoreMesh(core_axis_name="c", subcore_axis_name="s")

    @functools.partial(
        pl.kernel,
        mesh=mesh,
        compiler_params=_SC_PARAMS,
        out_type=jax.ShapeDtypeStruct((GGG, 16), jnp.float32),
        scratch_types=[
            pltpu.VMEM((NBUF, NCHAN, CELLS + 16), jnp.float32),
            pltpu.VMEM((NBUF, CELLS, 16), jnp.float32),
            pltpu.SemaphoreType.DMA,
            pltpu.SemaphoreType.DMA,
        ],
    )
    def k(h_hbm, e_hbm, tbl_hbm, cbuf, tbuf, sem_in, sem_out):
        wid = lax.axis_index("s") * info.num_cores + lax.axis_index("c")
        tile_base = wid * per_tile
        lane = lax.iota(jnp.int32, 16)

        def in_copies(it):
            slot = lax.rem(it, NBUF)
            base = tile_base + it * CELLS
            cps = [pltpu.make_async_copy(
                h_hbm.at[pl.ds(base, CELLS)],
                cbuf.at[slot, 0, pl.ds(0, CELLS)], sem_in)]
            for c in range(4):
                cps.append(pltpu.make_async_copy(
                    e_hbm.at[pl.ds(c * GGG + base, CELLS)],
                    cbuf.at[slot, c + 1, pl.ds(0, CELLS)], sem_in))
            return cps

        def out_copies(it):
            slot = lax.rem(it, NBUF)
            base = tile_base + it * CELLS
            return [pltpu.make_async_copy(
                tbuf.at[slot], tbl_hbm.at[pl.ds(base, CELLS)], sem_out)]

        def compute(it):
            slot = lax.rem(it, NBUF)

            def body(gg, carry):
                i = gg * 16
                cellv = i + lane
                is_last = jnp.logical_and(lane == 15, lax.rem(gg, 8) == 7)
                for c in range(NCHAN):
                    v = cbuf[slot, c, pl.ds(i, 16)]
                    vs = cbuf[slot, c, pl.ds(i + 1, 16)]
                    vs = jnp.where(is_last, v, vs)
                    plsc.store_scatter(
                        tbuf.at[slot],
                        [cellv, jnp.full((16,), c, jnp.int32)], v)
                    plsc.store_scatter(
                        tbuf.at[slot],
                        [cellv, jnp.full((16,), 8 + c, jnp.int32)], vs)
                return carry

            lax.fori_loop(0, CELLS // 16, body, 0)

        for cp in in_copies(0):
            cp.start()
        if n_chunks > 1:
            for cp in in_copies(1):
                cp.start()

        def loop_body(it, carry):
            for cp in in_copies(it):
                cp.wait()

            @pl.when(it >= 2)
            def _():
                for cp in out_copies(it - 2):
                    cp.wait()

            compute(it)
            for cp in out_copies(it):
                cp.start()

            @pl.when(it + 2 < n_chunks)
            def _():
                for cp in in_copies(it + 2):
                    cp.start()

            return carry

        lax.fori_loop(0, n_chunks, loop_body, 0)
        for cp in out_copies(n_chunks - 2):
            cp.wait()
        for cp in out_copies(n_chunks - 1):
            cp.wait()

    return k(hflat, eflat)


def _sc_sample(posflat, table, n_points):
    info = plsc.get_sparse_core_info()
    nc, ns = info.num_cores, info.num_subcores
    nw = nc * ns
    per_tile = n_points // nw
    n_chunks = per_tile // CHUNK

    mesh = plsc.VectorSubcoreMesh(core_axis_name="c", subcore_axis_name="s")

    @functools.partial(
        pl.kernel,
        mesh=mesh,
        compiler_params=_SC_PARAMS,
        out_type=[
            jax.ShapeDtypeStruct((n_points,), jnp.float32),
            jax.ShapeDtypeStruct((n_points * 4,), jnp.float32),
        ],
        scratch_types=[
            pltpu.VMEM((NBUF, 3 * CHUNK), jnp.float32),
            pltpu.VMEM((NBUF, 3, CHUNK), jnp.float32),
            pltpu.VMEM((NBUF, 4, CHUNK), jnp.int32),
            pltpu.VMEM((NBUF, 4, CHUNK, 16), jnp.float32),
            pltpu.VMEM((NBUF, CHUNK), jnp.float32),
            pltpu.VMEM((NBUF, CHUNK * 4), jnp.float32),
            pltpu.SemaphoreType.DMA,
            pltpu.SemaphoreType.DMA,
            pltpu.SemaphoreType.DMA,
        ],
    )
    def k(pos_hbm, table_hbm, outh_hbm, oute_hbm,
          posbuf, wbuf, idxbuf, gbuf, obufh, obufe, sem_pos, sem_g, sem_out):
        wid = lax.axis_index("s") * nc + lax.axis_index("c")
        tile_base = wid * per_tile
        lane = lax.iota(jnp.int32, 16)

        def pos_copies(it):
            slot = lax.rem(it, NBUF)
            base = tile_base + it * CHUNK
            return [pltpu.make_async_copy(
                pos_hbm.at[pl.ds(base * 3, CHUNK * 3)],
                posbuf.at[slot], sem_pos)]

        def gather_copies(it):
            slot = lax.rem(it, NBUF)
            return [pltpu.make_async_copy(table_hbm.at[idxbuf.at[slot, kk]],
                                          gbuf.at[slot, kk], sem_g)
                    for kk in range(4)]

        def out_copies(it):
            slot = lax.rem(it, NBUF)
            base = tile_base + it * CHUNK
            return [
                pltpu.make_async_copy(obufh.at[slot],
                                      outh_hbm.at[pl.ds(base, CHUNK)],
                                      sem_out),
                pltpu.make_async_copy(obufe.at[slot],
                                      oute_hbm.at[pl.ds(base * 4, CHUNK * 4)],
                                      sem_out),
            ]

        def phase1(it):
            slot = lax.rem(it, NBUF)

            def body(g, carry):
                p = g * 16
                pidx = (p + lane) * 3
                px = plsc.load_gather(posbuf.at[slot], [pidx])
                py = plsc.load_gather(posbuf.at[slot], [pidx + 1])
                pz = plsc.load_gather(posbuf.at[slot], [pidx + 2])
                x = jnp.clip((px + 1.0) * (0.5 * (G - 1)), 0.0, float(G - 1))
                y = jnp.clip((py + 1.0) * (0.5 * (G - 1)), 0.0, float(G - 1))
                z = jnp.clip((pz + 1.0) * (0.5 * (G - 1)), 0.0, float(G - 1))
                xi = x.astype(jnp.int32)
                yi = y.astype(jnp.int32)
                zi = z.astype(jnp.int32)
                y1 = jnp.minimum(yi + 1, G - 1)
                z1 = jnp.minimum(zi + 1, G - 1)
                idxbuf[slot, 0, pl.ds(p, 16)] = (zi * G + yi) * G + xi
                idxbuf[slot, 1, pl.ds(p, 16)] = (zi * G + y1) * G + xi
                idxbuf[slot, 2, pl.ds(p, 16)] = (z1 * G + yi) * G + xi
                idxbuf[slot, 3, pl.ds(p, 16)] = (z1 * G + y1) * G + xi
                wbuf[slot, 0, pl.ds(p, 16)] = x - xi.astype(jnp.float32)
                wbuf[slot, 1, pl.ds(p, 16)] = y - yi.astype(jnp.float32)
                wbuf[slot, 2, pl.ds(p, 16)] = z - zi.astype(jnp.float32)
                return carry

            lax.fori_loop(0, NGROUPS, body, 0)

        def phase2(it):
            slot = lax.rem(it, NBUF)

            def body(g, carry):
                p = g * 16
                wx = wbuf[slot, 0, pl.ds(p, 16)]
                wy = wbuf[slot, 1, pl.ds(p, 16)]
                wz = wbuf[slot, 2, pl.ds(p, 16)]
                ux = 1.0 - wx
                wk = ((1.0 - wz) * (1.0 - wy), (1.0 - wz) * wy,
                      wz * (1.0 - wy), wz * wy)
                rowv = p + lane
                slotv = jnp.full((16,), slot, jnp.int32)
                for c in range(NCHAN):
                    acc = None
                    for kk in range(4):
                        kv = jnp.full((16,), kk, jnp.int32)
                        v0 = plsc.load_gather(
                            gbuf,
                            [slotv, kv, rowv, jnp.full((16,), c, jnp.int32)])
                        v1 = plsc.load_gather(
                            gbuf,
                            [slotv, kv, rowv,
                             jnp.full((16,), 8 + c, jnp.int32)])
                        term = wk[kk] * (v0 * ux + v1 * wx)
                        acc = term if acc is None else acc + term
                    if c == 0:
                        obufh[slot, pl.ds(p, 16)] = acc
                    else:
                        plsc.store_scatter(obufe.at[slot],
                                           [rowv * 4 + (c - 1)], acc)
                return carry

            lax.fori_loop(0, NGROUPS, body, 0)

        for cp in pos_copies(0):
            cp.start()
        if n_chunks > 1:
            for cp in pos_copies(1):
                cp.start()

        def loop_body(it, carry):
            for cp in pos_copies(it):
                cp.wait()
            phase1(it)
            for cp in gather_copies(it):
                cp.start()

            @pl.when(it + 2 < n_chunks)
            def _():
                for cp in pos_copies(it + 2):
                    cp.start()

            @pl.when(it >= 1)
            def _():
                for cp in gather_copies(it - 1):
                    cp.wait()

                @pl.when(it >= 3)
                def _():
                    for cp in out_copies(it - 3):
                        cp.wait()

                phase2(it - 1)
                for cp in out_copies(it - 1):
                    cp.start()

            return carry

        lax.fori_loop(0, n_chunks, loop_body, 0)

        last = n_chunks - 1
        for cp in gather_copies(last):
            cp.wait()
        if n_chunks >= 3:
            for cp in out_copies(last - 2):
                cp.wait()
        phase2(last)
        for cp in out_copies(last):
            cp.start()
        if n_chunks >= 2:
            for cp in out_copies(last - 1):
                cp.wait()
        for cp in out_copies(last):
            cp.wait()

    return k(posflat, table)


def kernel(positions, H, E):
    Bb, Tt, _ = positions.shape
    n_points = Bb * Tt
    table = _build_table(H.reshape(GGG), E.reshape(4 * GGG))
    posflat = positions.reshape(n_points * 3)
    outh, oute = _sc_sample(posflat, table, n_points)
    return outh.reshape(Bb, Tt), oute.reshape(Bb, Tt, 4)
